# baseline (speedup 1.0000x reference)
import functools

import jax
import jax.numpy as jnp
from jax import lax
from jax.experimental import pallas as pl
from jax.experimental.pallas import tpu as pltpu
from jax.experimental.pallas import tpu_sc as plsc

VOCAB = 100000
D = 128
P = 20
B = 4096
S = 200
T = S - P
TP = 184

NC = 2
NS = 16
NW = NC * NS
BPW = B // NW

NBUF = 3
AHEAD = 2

_mesh = plsc.VectorSubcoreMesh(core_axis_name="c", subcore_axis_name="s")


@functools.partial(
    pl.kernel,
    mesh=_mesh,
    compiler_params=pltpu.CompilerParams(use_tc_tiling_on_sc=False),
    out_type=jax.ShapeDtypeStruct((B, S, D), jnp.float32),
    scratch_types=[
        pltpu.VMEM((BPW, TP), jnp.int32),
        pltpu.VMEM((S + 4, D), jnp.float32),
        pltpu.VMEM((S + 4, D), jnp.float32),
        pltpu.VMEM((S + 4, D), jnp.float32),
        pltpu.SemaphoreType.DMA,
        pltpu.SemaphoreType.DMA,
        pltpu.SemaphoreType.DMA,
        pltpu.SemaphoreType.DMA,
        pltpu.SemaphoreType.DMA,
        pltpu.SemaphoreType.DMA,
    ],
)
def _embed_kernel(inp_hbm, table_hbm, cp_hbm, out_hbm, idx_all,
                  buf0, buf1, buf2, gsem0, gsem1, gsem2,
                  ssem0, ssem1, ssem2):
    bufs = (buf0, buf1, buf2)
    gsems = (gsem0, gsem1, gsem2)
    ssems = (ssem0, ssem1, ssem2)
    wid = lax.axis_index("s") * NC + lax.axis_index("c")
    base = wid * BPW

    pltpu.sync_copy(inp_hbm.at[pl.ds(base, BPW), pl.ds(0, TP)], idx_all)

    for buf in bufs:
        pltpu.sync_copy(cp_hbm, buf.at[pl.ds(0, P)])

    def fire_gathers(j, k):
        pltpu.async_copy(
            table_hbm.at[idx_all.at[j, pl.ds(0, 128)]],
            bufs[k].at[pl.ds(P, 128)], gsems[k],
        )
        pltpu.async_copy(
            table_hbm.at[idx_all.at[j, pl.ds(128, TP - 128)]],
            bufs[k].at[pl.ds(P + 128, TP - 128)], gsems[k],
        )

    def wait_gathers(j, k):
        pltpu.make_async_copy(
            table_hbm.at[idx_all.at[j, pl.ds(0, 128)]],
            bufs[k].at[pl.ds(P, 128)], gsems[k],
        ).wait()
        pltpu.make_async_copy(
            table_hbm.at[idx_all.at[j, pl.ds(128, TP - 128)]],
            bufs[k].at[pl.ds(P + 128, TP - 128)], gsems[k],
        ).wait()

    def wait_store(j, k):
        pltpu.make_async_copy(
            bufs[k].at[pl.ds(0, S)], out_hbm.at[base + j], ssems[k],
        ).wait()

    for j0 in range(AHEAD):
        fire_gathers(j0, j0 % NBUF)

    def body(i, carry):
        for k in range(NBUF):
            j = NBUF * i + k
            wait_gathers(j, k)
            pltpu.async_copy(
                bufs[k].at[pl.ds(0, S)], out_hbm.at[base + j], ssems[k],
            )
            jj = j + AHEAD
            kk = (k + AHEAD) % NBUF

            @pl.when(jj < BPW)
            def _():
                @pl.when(jj >= NBUF)
                def _():
                    wait_store(jj - NBUF, kk)

                fire_gathers(jj, kk)

        return carry

    lax.fori_loop(0, BPW // NBUF, body, 0)

    for r in range((BPW // NBUF) * NBUF, BPW):
        k = r % NBUF
        wait_gathers(r, k)
        pltpu.async_copy(
            bufs[k].at[pl.ds(0, S)], out_hbm.at[base + r], ssems[k],
        )

    for j in range(BPW - NBUF, BPW):
        wait_store(j, j % NBUF)


def kernel(input, table, cp):
    return _embed_kernel(input.astype(jnp.int32), table, cp)


# speedup vs baseline: 1.0045x; 1.0045x over previous
import functools

import jax
import jax.numpy as jnp
from jax import lax
from jax.experimental import pallas as pl
from jax.experimental.pallas import tpu as pltpu
from jax.experimental.pallas import tpu_sc as plsc

VOCAB = 100000
D = 128
P = 20
B = 4096
S = 200
T = S - P
TP = 184

NC = 2
NS = 16
NW = NC * NS
BPW = B // NW

NBUF = 3
AHEAD = 2

_mesh = plsc.VectorSubcoreMesh(core_axis_name="c", subcore_axis_name="s")


@functools.partial(
    pl.kernel,
    mesh=_mesh,
    compiler_params=pltpu.CompilerParams(use_tc_tiling_on_sc=False),
    out_type=jax.ShapeDtypeStruct((B, S, D), jnp.float32),
    scratch_types=[
        pltpu.VMEM((BPW, S), jnp.int32),
        pltpu.VMEM((S + 4, D), jnp.float32),
        pltpu.VMEM((S + 4, D), jnp.float32),
        pltpu.VMEM((S + 4, D), jnp.float32),
        pltpu.SemaphoreType.DMA,
        pltpu.SemaphoreType.DMA,
        pltpu.SemaphoreType.DMA,
        pltpu.SemaphoreType.DMA,
        pltpu.SemaphoreType.DMA,
        pltpu.SemaphoreType.DMA,
    ],
)
def _embed_kernel(inp_hbm, table_hbm, cp_hbm, out_hbm, idx_all,
                  buf0, buf1, buf2, gsem0, gsem1, gsem2,
                  ssem0, ssem1, ssem2):
    bufs = (buf0, buf1, buf2)
    gsems = (gsem0, gsem1, gsem2)
    ssems = (ssem0, ssem1, ssem2)
    wid = lax.axis_index("s") * NC + lax.axis_index("c")
    base = wid * BPW

    idx_cp = pltpu.async_copy(inp_hbm.at[pl.ds(base, BPW)], idx_all, gsems[0])
    cp_cps = [pltpu.async_copy(cp_hbm, buf.at[pl.ds(0, P)], ssems[i])
              for i, buf in enumerate(bufs)]
    idx_cp.wait()
    for c in cp_cps:
        c.wait()

    def fire_gathers(j, k):
        pltpu.async_copy(
            table_hbm.at[idx_all.at[j, pl.ds(0, 128)]],
            bufs[k].at[pl.ds(P, 128)], gsems[k],
        )
        pltpu.async_copy(
            table_hbm.at[idx_all.at[j, pl.ds(128, TP - 128)]],
            bufs[k].at[pl.ds(P + 128, TP - 128)], gsems[k],
        )

    def wait_gathers(j, k):
        pltpu.make_async_copy(
            table_hbm.at[idx_all.at[j, pl.ds(0, 128)]],
            bufs[k].at[pl.ds(P, 128)], gsems[k],
        ).wait()
        pltpu.make_async_copy(
            table_hbm.at[idx_all.at[j, pl.ds(128, TP - 128)]],
            bufs[k].at[pl.ds(P + 128, TP - 128)], gsems[k],
        ).wait()

    def wait_store(j, k):
        pltpu.make_async_copy(
            bufs[k].at[pl.ds(0, S)], out_hbm.at[base + j], ssems[k],
        ).wait()

    for j0 in range(AHEAD):
        fire_gathers(j0, j0 % NBUF)

    def body(i, carry):
        for k in range(NBUF):
            j = NBUF * i + k
            wait_gathers(j, k)
            pltpu.async_copy(
                bufs[k].at[pl.ds(0, S)], out_hbm.at[base + j], ssems[k],
            )
            jj = j + AHEAD
            kk = (k + AHEAD) % NBUF

            @pl.when(jj < BPW)
            def _():
                @pl.when(jj >= NBUF)
                def _():
                    wait_store(jj - NBUF, kk)

                fire_gathers(jj, kk)

        return carry

    lax.fori_loop(0, BPW // NBUF, body, 0)

    for r in range((BPW // NBUF) * NBUF, BPW):
        k = r % NBUF
        wait_gathers(r, k)
        pltpu.async_copy(
            bufs[k].at[pl.ds(0, S)], out_hbm.at[base + r], ssems[k],
        )

    for j in range(BPW - NBUF, BPW):
        wait_store(j, j % NBUF)


def kernel(input, table, cp):
    return _embed_kernel(input.astype(jnp.int32), table, cp)
